# baseline (speedup 1.0000x reference)
import functools

import jax
import jax.numpy as jnp
from jax import lax
from jax.experimental import pallas as pl
from jax.experimental.pallas import tpu as pltpu
from jax.experimental.pallas import tpu_sc as plsc

_N_E = 8192
_E_DIM = 32
_BETA = 0.25
_B_ROWS = 16384
_W = 4096
_MB = 512

_NW = 32
_BPW = _B_ROWS // _NW
_GCH = 128


def _tree_sum_sublanes(s):
    t = s[0:16, :] + s[16:32, :]
    t = t[0:8, :] + t[8:16, :]
    t = t[0:4, :] + t[4:8, :]
    t = t[0:2, :] + t[2:4, :]
    return t[0:1, :] + t[1:2, :]


def _tree_sum_lanes(s):
    t = s[:, 0:16] + s[:, 16:32]
    t = t[:, 0:8] + t[:, 8:16]
    t = t[:, 0:4] + t[:, 4:8]
    t = t[:, 0:2] + t[:, 2:4]
    return t[:, 0:1] + t[:, 1:2]


def _argmin_body(zt_ref, emb_ref, idx_ref):
    x = zt_ref[...]
    zs = _tree_sum_sublanes(x * x)
    xb = x.astype(jnp.bfloat16)
    state_v = jnp.full((1, _MB), jnp.inf, jnp.float32)
    state_i = jnp.zeros((1, _MB), jnp.int32)
    for w in range(_N_E // _W):
        e = emb_ref[pl.ds(w * _W, _W), :]
        es = _tree_sum_lanes(e * e)
        eb = e.astype(jnp.bfloat16)
        mm = lax.dot_general(eb, xb, (((1,), (0,)), ((), ())),
                             preferred_element_type=jnp.float32)
        d = (zs + es) - 2.0 * mm
        m = jnp.min(d, axis=0, keepdims=True)
        ii = lax.broadcasted_iota(jnp.int32, (_W, _MB), 0) + w * _W
        cand = jnp.where(d == m, ii, jnp.int32(2 ** 30))
        i_w = jnp.min(cand, axis=0, keepdims=True)
        take = (m < state_v) | ((m == state_v) & (i_w < state_i))
        state_i = jnp.where(take, i_w, state_i)
        state_v = jnp.where(take, m, state_v)
        state_v = state_v.astype(jnp.bfloat16).astype(jnp.float32)
    idx_ref[...] = state_i


def _argmin_indices(zt, emb):
    return pl.pallas_call(
        _argmin_body,
        grid=(_B_ROWS // _MB,),
        in_specs=[pl.BlockSpec((_E_DIM, _MB), lambda i: (0, i)),
                  pl.BlockSpec((_N_E, _E_DIM), lambda i: (0, 0))],
        out_specs=pl.BlockSpec((1, _MB), lambda i: (0, i)),
        out_shape=jax.ShapeDtypeStruct((1, _B_ROWS), jnp.int32),
    )(zt, emb)


def _make_sc_kernel():
    mesh = plsc.VectorSubcoreMesh(core_axis_name="c", subcore_axis_name="s")

    @functools.partial(
        pl.kernel, mesh=mesh,
        compiler_params=pltpu.CompilerParams(needs_layout_passes=False,
                                             use_tc_tiling_on_sc=False),
        out_type=[jax.ShapeDtypeStruct((_B_ROWS, _E_DIM), jnp.float32),
                  jax.ShapeDtypeStruct((_N_E,), jnp.float32),
                  jax.ShapeDtypeStruct((_NW, 16), jnp.float32),
                  jax.ShapeDtypeStruct((16,), jnp.float32)],
        scratch_types=[pltpu.VMEM((_BPW,), jnp.int32),
                       pltpu.VMEM((_BPW, _E_DIM), jnp.float32),
                       pltpu.VMEM((_BPW, _E_DIM), jnp.float32),
                       pltpu.VMEM((16,), jnp.float32),
                       pltpu.VMEM((_B_ROWS,), jnp.int32),
                       pltpu.VMEM((_N_E,), jnp.float32),
                       pltpu.VMEM((16,), jnp.float32),
                       pltpu.SemaphoreType.DMA],
    )
    def sc_kernel(idx_hbm, emb_hbm, zp_hbm,
                  zq_hbm, usage_hbm, lp_hbm, up_hbm,
                  idx_v, rows_v, zp_v, acc_v, idxall_v, usage_v, uacc_v, sem):
        wid = lax.axis_index("s") * 2 + lax.axis_index("c")
        base = wid * _BPW
        pltpu.sync_copy(idx_hbm.at[pl.ds(base, _BPW)], idx_v)
        for g in range(_BPW // _GCH):
            pltpu.async_copy(
                emb_hbm.at[idx_v.at[pl.ds(g * _GCH, _GCH)]],
                rows_v.at[pl.ds(g * _GCH, _GCH)], sem)
        pltpu.sync_copy(zp_hbm.at[pl.ds(base, _BPW)], zp_v)
        for g in range(_BPW // _GCH):
            pltpu.make_async_copy(
                emb_hbm.at[idx_v.at[pl.ds(g * _GCH, _GCH)]],
                rows_v.at[pl.ds(g * _GCH, _GCH)], sem).wait()

        def row_step(i, acc):
            for h in range(_E_DIM // 16):
                g = rows_v[i, pl.ds(h * 16, 16)]
                p = zp_v[i, pl.ds(h * 16, 16)]
                df = g - p
                acc = acc + df * df
                rows_v[i, pl.ds(h * 16, 16)] = p + df
            return acc

        acc = lax.fori_loop(0, _BPW, row_step, jnp.zeros((16,), jnp.float32))
        acc_v[...] = acc
        pltpu.sync_copy(rows_v, zq_hbm.at[pl.ds(base, _BPW)])
        pltpu.sync_copy(acc_v, lp_hbm.at[wid])

        @pl.when(wid == 0)
        def _():
            pltpu.sync_copy(idx_hbm, idxall_v)

            def z_step(i, _):
                usage_v[pl.ds(i * 16, 16)] = jnp.zeros((16,), jnp.float32)
                return 0
            lax.fori_loop(0, _N_E // 16, z_step, 0)

            ones = jnp.ones((16,), jnp.float32)

            def s_step(j, _):
                iv = idxall_v[pl.ds(j * 16, 16)]
                plsc.store_scatter(usage_v, [iv], ones)
                return 0
            lax.fori_loop(0, _B_ROWS // 16, s_step, 0)

            def u_step(i, uacc):
                return uacc + usage_v[pl.ds(i * 16, 16)]
            uacc = lax.fori_loop(0, _N_E // 16, u_step,
                                 jnp.zeros((16,), jnp.float32))
            uacc_v[...] = uacc
            pltpu.sync_copy(usage_v, usage_hbm)
            pltpu.sync_copy(uacc_v, up_hbm)

    return sc_kernel


_sc_kernel = _make_sc_kernel()


def kernel(z, emb):
    zt = z.transpose(1, 0, 2, 3).reshape(_E_DIM, _B_ROWS)
    zf = z.transpose(0, 2, 3, 1).reshape(_B_ROWS, _E_DIM)

    idx2d = _argmin_indices(zt, emb)
    idx = idx2d.reshape(_B_ROWS)

    zq_flat, usage, lp, up = _sc_kernel(idx, emb, zf)

    z_q_out = zq_flat.reshape(16, 32, 32, _E_DIM).transpose(0, 3, 1, 2)
    m = jnp.sum(lp) / jnp.float32(_B_ROWS * _E_DIM)
    loss = m + jnp.float32(_BETA) * m
    uni = jnp.sum(up)
    num_unique = uni.astype(jnp.int32)
    total_usage = uni / jnp.float32(_N_E)
    return (z_q_out, loss, idx, num_unique, usage, total_usage)


# speedup vs baseline: 1.3193x; 1.3193x over previous
import functools

import jax
import jax.numpy as jnp
from jax import lax
from jax.experimental import pallas as pl
from jax.experimental.pallas import tpu as pltpu
from jax.experimental.pallas import tpu_sc as plsc

_N_E = 8192
_E_DIM = 32
_BETA = 0.25
_B_ROWS = 16384
_W = 4096
_MB = 1024

_NW = 32
_BPW = _B_ROWS // _NW
_GCH = 128


def _tree_sum_sublanes(s):
    t = s[0:16, :] + s[16:32, :]
    t = t[0:8, :] + t[8:16, :]
    t = t[0:4, :] + t[4:8, :]
    t = t[0:2, :] + t[2:4, :]
    return t[0:1, :] + t[1:2, :]


def _tree_sum_lanes(s):
    t = s[:, 0:16] + s[:, 16:32]
    t = t[:, 0:8] + t[:, 8:16]
    t = t[:, 0:4] + t[:, 4:8]
    t = t[:, 0:2] + t[:, 2:4]
    return t[:, 0:1] + t[:, 1:2]


def _argmin_body(zt_ref, emb_ref, idx_ref):
    x = zt_ref[...]
    zs = _tree_sum_sublanes(x * x)
    xb = x.astype(jnp.bfloat16)
    state_v = jnp.full((1, _MB), jnp.inf, jnp.float32)
    state_i = jnp.zeros((1, _MB), jnp.int32)
    ii = lax.broadcasted_iota(jnp.int32, (_W, _MB), 0)
    for w in range(_N_E // _W):
        e = emb_ref[pl.ds(w * _W, _W), :]
        es = _tree_sum_lanes(e * e)
        eb = (jnp.float32(-2.0) * e).astype(jnp.bfloat16)
        mm2 = lax.dot_general(eb, xb, (((1,), (0,)), ((), ())),
                              preferred_element_type=jnp.float32)
        d = (zs + es) + mm2
        m = jnp.min(d, axis=0, keepdims=True)
        cand = jnp.where(d == m, ii, jnp.int32(2 ** 30))
        i_w = jnp.min(cand, axis=0, keepdims=True) + w * _W
        take = m < state_v
        state_i = jnp.where(take, i_w, state_i)
        state_v = jnp.where(take, m, state_v)
        state_v = state_v.astype(jnp.bfloat16).astype(jnp.float32)
    idx_ref[...] = state_i


def _argmin_indices(zt, emb):
    return pl.pallas_call(
        _argmin_body,
        grid=(_B_ROWS // _MB,),
        in_specs=[pl.BlockSpec((_E_DIM, _MB), lambda i: (0, i)),
                  pl.BlockSpec((_N_E, _E_DIM), lambda i: (0, 0))],
        out_specs=pl.BlockSpec((1, _MB), lambda i: (0, i)),
        out_shape=jax.ShapeDtypeStruct((1, _B_ROWS), jnp.int32),
    )(zt, emb)


def _make_sc_kernel():
    mesh = plsc.VectorSubcoreMesh(core_axis_name="c", subcore_axis_name="s")

    @functools.partial(
        pl.kernel, mesh=mesh,
        compiler_params=pltpu.CompilerParams(needs_layout_passes=False,
                                             use_tc_tiling_on_sc=False),
        out_type=[jax.ShapeDtypeStruct((_B_ROWS, _E_DIM), jnp.float32),
                  jax.ShapeDtypeStruct((_N_E,), jnp.float32),
                  jax.ShapeDtypeStruct((_NW, 16), jnp.float32),
                  jax.ShapeDtypeStruct((16,), jnp.float32)],
        scratch_types=[pltpu.VMEM((_BPW,), jnp.int32),
                       pltpu.VMEM((_BPW, _E_DIM), jnp.float32),
                       pltpu.VMEM((_BPW, _E_DIM), jnp.float32),
                       pltpu.VMEM((16,), jnp.float32),
                       pltpu.VMEM((_B_ROWS,), jnp.int32),
                       pltpu.VMEM((_N_E,), jnp.float32),
                       pltpu.VMEM((16,), jnp.float32),
                       pltpu.SemaphoreType.DMA],
    )
    def sc_kernel(idx_hbm, emb_hbm, zp_hbm,
                  zq_hbm, usage_hbm, lp_hbm, up_hbm,
                  idx_v, rows_v, zp_v, acc_v, idxall_v, usage_v, uacc_v, sem):
        wid = lax.axis_index("s") * 2 + lax.axis_index("c")
        base = wid * _BPW
        pltpu.sync_copy(idx_hbm.at[pl.ds(base, _BPW)], idx_v)
        for g in range(_BPW // _GCH):
            pltpu.async_copy(
                emb_hbm.at[idx_v.at[pl.ds(g * _GCH, _GCH)]],
                rows_v.at[pl.ds(g * _GCH, _GCH)], sem)
        pltpu.sync_copy(zp_hbm.at[pl.ds(base, _BPW)], zp_v)
        for g in range(_BPW // _GCH):
            pltpu.make_async_copy(
                emb_hbm.at[idx_v.at[pl.ds(g * _GCH, _GCH)]],
                rows_v.at[pl.ds(g * _GCH, _GCH)], sem).wait()

        def row_step(i, acc):
            for h in range(_E_DIM // 16):
                g = rows_v[i, pl.ds(h * 16, 16)]
                p = zp_v[i, pl.ds(h * 16, 16)]
                df = g - p
                acc = acc + df * df
                rows_v[i, pl.ds(h * 16, 16)] = p + df
            return acc

        acc = lax.fori_loop(0, _BPW, row_step, jnp.zeros((16,), jnp.float32))
        acc_v[...] = acc
        pltpu.sync_copy(rows_v, zq_hbm.at[pl.ds(base, _BPW)])
        pltpu.sync_copy(acc_v, lp_hbm.at[wid])

        @pl.when(wid == 0)
        def _():
            pltpu.sync_copy(idx_hbm, idxall_v)

            def z_step(i, _):
                usage_v[pl.ds(i * 16, 16)] = jnp.zeros((16,), jnp.float32)
                return 0
            lax.fori_loop(0, _N_E // 16, z_step, 0)

            ones = jnp.ones((16,), jnp.float32)

            def s_step(j, _):
                iv = idxall_v[pl.ds(j * 16, 16)]
                plsc.store_scatter(usage_v, [iv], ones)
                return 0
            lax.fori_loop(0, _B_ROWS // 16, s_step, 0)

            def u_step(i, uacc):
                return uacc + usage_v[pl.ds(i * 16, 16)]
            uacc = lax.fori_loop(0, _N_E // 16, u_step,
                                 jnp.zeros((16,), jnp.float32))
            uacc_v[...] = uacc
            pltpu.sync_copy(usage_v, usage_hbm)
            pltpu.sync_copy(uacc_v, up_hbm)

    return sc_kernel


_sc_kernel = _make_sc_kernel()


def kernel(z, emb):
    zt = z.transpose(1, 0, 2, 3).reshape(_E_DIM, _B_ROWS)
    zf = z.transpose(0, 2, 3, 1).reshape(_B_ROWS, _E_DIM)

    idx2d = _argmin_indices(zt, emb)
    idx = idx2d.reshape(_B_ROWS)

    zq_flat, usage, lp, up = _sc_kernel(idx, emb, zf)

    z_q_out = zq_flat.reshape(16, 32, 32, _E_DIM).transpose(0, 3, 1, 2)
    m = jnp.sum(lp) / jnp.float32(_B_ROWS * _E_DIM)
    loss = m + jnp.float32(_BETA) * m
    uni = jnp.sum(up)
    num_unique = uni.astype(jnp.int32)
    total_usage = uni / jnp.float32(_N_E)
    return (z_q_out, loss, idx, num_unique, usage, total_usage)


# speedup vs baseline: 1.3813x; 1.0470x over previous
import functools

import jax
import jax.numpy as jnp
from jax import lax
from jax.experimental import pallas as pl
from jax.experimental.pallas import tpu as pltpu
from jax.experimental.pallas import tpu_sc as plsc

_N_E = 8192
_E_DIM = 32
_BETA = 0.25
_B_ROWS = 16384
_W = 4096
_MB = 1024

_NW = 32
_BPW = _B_ROWS // _NW
_GCH = 128


def _tree_sum_sublanes(s):
    t = s[0:16, :] + s[16:32, :]
    t = t[0:8, :] + t[8:16, :]
    t = t[0:4, :] + t[4:8, :]
    t = t[0:2, :] + t[2:4, :]
    return t[0:1, :] + t[1:2, :]


def _tree_sum_lanes(s):
    t = s[:, 0:16] + s[:, 16:32]
    t = t[:, 0:8] + t[:, 8:16]
    t = t[:, 0:4] + t[:, 4:8]
    t = t[:, 0:2] + t[:, 2:4]
    return t[:, 0:1] + t[:, 1:2]


def _argmin_body(zf_ref, emb_ref, idx_ref):
    x = jnp.transpose(zf_ref[...], (1, 0))
    zs = _tree_sum_sublanes(x * x)
    xb = x.astype(jnp.bfloat16)
    state_v = jnp.full((1, _MB), jnp.inf, jnp.float32)
    state_i = jnp.zeros((1, _MB), jnp.int32)
    ii = lax.broadcasted_iota(jnp.int32, (_W, _MB), 0)
    for w in range(_N_E // _W):
        e = emb_ref[pl.ds(w * _W, _W), :]
        es = _tree_sum_lanes(e * e)
        eb = (jnp.float32(-2.0) * e).astype(jnp.bfloat16)
        mm2 = lax.dot_general(eb, xb, (((1,), (0,)), ((), ())),
                              preferred_element_type=jnp.float32)
        d = (zs + es) + mm2
        m = jnp.min(d, axis=0, keepdims=True)
        cand = jnp.where(d == m, ii, jnp.int32(2 ** 30))
        i_w = jnp.min(cand, axis=0, keepdims=True) + w * _W
        take = m < state_v
        state_i = jnp.where(take, i_w, state_i)
        state_v = jnp.where(take, m, state_v)
        state_v = state_v.astype(jnp.bfloat16).astype(jnp.float32)
    idx_ref[...] = state_i


def _argmin_indices(zf, emb):
    return pl.pallas_call(
        _argmin_body,
        grid=(_B_ROWS // _MB,),
        in_specs=[pl.BlockSpec((_MB, _E_DIM), lambda i: (i, 0)),
                  pl.BlockSpec((_N_E, _E_DIM), lambda i: (0, 0))],
        out_specs=pl.BlockSpec((1, _MB), lambda i: (0, i)),
        out_shape=jax.ShapeDtypeStruct((1, _B_ROWS), jnp.int32),
    )(zf, emb)


def _make_sc_kernel():
    mesh = plsc.VectorSubcoreMesh(core_axis_name="c", subcore_axis_name="s")

    @functools.partial(
        pl.kernel, mesh=mesh,
        compiler_params=pltpu.CompilerParams(needs_layout_passes=False,
                                             use_tc_tiling_on_sc=False),
        out_type=[jax.ShapeDtypeStruct((_B_ROWS, _E_DIM), jnp.float32),
                  jax.ShapeDtypeStruct((_N_E,), jnp.float32),
                  jax.ShapeDtypeStruct((_NW, 16), jnp.float32),
                  jax.ShapeDtypeStruct((16,), jnp.float32)],
        scratch_types=[pltpu.VMEM((_BPW,), jnp.int32),
                       pltpu.VMEM((_BPW, _E_DIM), jnp.float32),
                       pltpu.VMEM((_BPW, _E_DIM), jnp.float32),
                       pltpu.VMEM((16,), jnp.float32),
                       pltpu.VMEM((_B_ROWS,), jnp.int32),
                       pltpu.VMEM((_N_E,), jnp.float32),
                       pltpu.VMEM((16,), jnp.float32),
                       pltpu.SemaphoreType.DMA],
    )
    def sc_kernel(idx_hbm, emb_hbm, zp_hbm,
                  zq_hbm, usage_hbm, lp_hbm, up_hbm,
                  idx_v, rows_v, zp_v, acc_v, idxall_v, usage_v, uacc_v, sem):
        wid = lax.axis_index("s") * 2 + lax.axis_index("c")
        base = wid * _BPW
        pltpu.sync_copy(idx_hbm.at[pl.ds(base, _BPW)], idx_v)
        for g in range(_BPW // _GCH):
            pltpu.async_copy(
                emb_hbm.at[idx_v.at[pl.ds(g * _GCH, _GCH)]],
                rows_v.at[pl.ds(g * _GCH, _GCH)], sem)
        pltpu.sync_copy(zp_hbm.at[pl.ds(base, _BPW)], zp_v)
        for g in range(_BPW // _GCH):
            pltpu.make_async_copy(
                emb_hbm.at[idx_v.at[pl.ds(g * _GCH, _GCH)]],
                rows_v.at[pl.ds(g * _GCH, _GCH)], sem).wait()

        def row_step(i, acc):
            for h in range(_E_DIM // 16):
                g = rows_v[i, pl.ds(h * 16, 16)]
                p = zp_v[i, pl.ds(h * 16, 16)]
                df = g - p
                acc = acc + df * df
                rows_v[i, pl.ds(h * 16, 16)] = p + df
            return acc

        acc = lax.fori_loop(0, _BPW, row_step, jnp.zeros((16,), jnp.float32))
        acc_v[...] = acc
        pltpu.sync_copy(rows_v, zq_hbm.at[pl.ds(base, _BPW)])
        pltpu.sync_copy(acc_v, lp_hbm.at[wid])

        @pl.when(wid == 0)
        def _():
            pltpu.sync_copy(idx_hbm, idxall_v)

            def z_step(i, _):
                usage_v[pl.ds(i * 16, 16)] = jnp.zeros((16,), jnp.float32)
                return 0
            lax.fori_loop(0, _N_E // 16, z_step, 0)

            ones = jnp.ones((16,), jnp.float32)

            def s_step(j, _):
                iv = idxall_v[pl.ds(j * 16, 16)]
                plsc.store_scatter(usage_v, [iv], ones)
                return 0
            lax.fori_loop(0, _B_ROWS // 16, s_step, 0)

            def u_step(i, uacc):
                return uacc + usage_v[pl.ds(i * 16, 16)]
            uacc = lax.fori_loop(0, _N_E // 16, u_step,
                                 jnp.zeros((16,), jnp.float32))
            uacc_v[...] = uacc
            pltpu.sync_copy(usage_v, usage_hbm)
            pltpu.sync_copy(uacc_v, up_hbm)

    return sc_kernel


_sc_kernel = _make_sc_kernel()


def kernel(z, emb):
    zf = z.transpose(0, 2, 3, 1).reshape(_B_ROWS, _E_DIM)

    idx2d = _argmin_indices(zf, emb)
    idx = idx2d.reshape(_B_ROWS)

    zq_flat, usage, lp, up = _sc_kernel(idx, emb, zf)

    z_q_out = zq_flat.reshape(16, 32, 32, _E_DIM).transpose(0, 3, 1, 2)
    m = jnp.sum(lp) / jnp.float32(_B_ROWS * _E_DIM)
    loss = m + jnp.float32(_BETA) * m
    uni = jnp.sum(up)
    num_unique = uni.astype(jnp.int32)
    total_usage = uni / jnp.float32(_N_E)
    return (z_q_out, loss, idx, num_unique, usage, total_usage)


# speedup vs baseline: 1.5654x; 1.1333x over previous
import functools

import jax
import jax.numpy as jnp
from jax import lax
from jax.experimental import pallas as pl
from jax.experimental.pallas import tpu as pltpu
from jax.experimental.pallas import tpu_sc as plsc

_N_E = 8192
_E_DIM = 32
_BETA = 0.25
_B_ROWS = 16384
_W = 4096
_MB = 1024

_NW = 32
_BPW = _B_ROWS // _NW
_GCH = 128


def _tree_sum_sublanes(s):
    t = s[0:16, :] + s[16:32, :]
    t = t[0:8, :] + t[8:16, :]
    t = t[0:4, :] + t[4:8, :]
    t = t[0:2, :] + t[2:4, :]
    return t[0:1, :] + t[1:2, :]


def _tree_sum_lanes(s):
    t = s[:, 0:16] + s[:, 16:32]
    t = t[:, 0:8] + t[:, 8:16]
    t = t[:, 0:4] + t[:, 4:8]
    t = t[:, 0:2] + t[:, 2:4]
    return t[:, 0:1] + t[:, 1:2]


def _argmin_body(zf_ref, emb_ref, idx_ref, es_ref, eb_ref):
    @pl.when(pl.program_id(0) == 0)
    def _():
        e0 = emb_ref[...]
        es_ref[...] = _tree_sum_lanes(e0 * e0)
        eb_ref[...] = (jnp.float32(-2.0) * e0).astype(jnp.bfloat16)

    x = jnp.transpose(zf_ref[...], (1, 0))
    zs = _tree_sum_sublanes(x * x)
    xb = x.astype(jnp.bfloat16)
    state_v = jnp.full((1, _MB), jnp.inf, jnp.float32)
    state_i = jnp.zeros((1, _MB), jnp.int32)
    ii = lax.broadcasted_iota(jnp.int32, (_W, _MB), 0)
    for w in range(_N_E // _W):
        es = es_ref[pl.ds(w * _W, _W), :]
        eb = eb_ref[pl.ds(w * _W, _W), :]
        mm2 = lax.dot_general(eb, xb, (((1,), (0,)), ((), ())),
                              preferred_element_type=jnp.float32)
        d = (zs + es) + mm2
        m = jnp.min(d, axis=0, keepdims=True)
        cand = jnp.where(d == m, ii, jnp.int32(2 ** 30))
        i_w = jnp.min(cand, axis=0, keepdims=True) + w * _W
        take = m < state_v
        state_i = jnp.where(take, i_w, state_i)
        state_v = jnp.where(take, m, state_v)
        state_v = state_v.astype(jnp.bfloat16).astype(jnp.float32)
    idx_ref[...] = state_i


def _argmin_indices(zf, emb):
    return pl.pallas_call(
        _argmin_body,
        grid=(_B_ROWS // _MB,),
        in_specs=[pl.BlockSpec((_MB, _E_DIM), lambda i: (i, 0)),
                  pl.BlockSpec((_N_E, _E_DIM), lambda i: (0, 0))],
        out_specs=pl.BlockSpec((1, _MB), lambda i: (0, i)),
        out_shape=jax.ShapeDtypeStruct((1, _B_ROWS), jnp.int32),
        scratch_shapes=[pltpu.VMEM((_N_E, 1), jnp.float32),
                        pltpu.VMEM((_N_E, _E_DIM), jnp.bfloat16)],
    )(zf, emb)


def _make_sc_kernel():
    mesh = plsc.VectorSubcoreMesh(core_axis_name="c", subcore_axis_name="s")

    @functools.partial(
        pl.kernel, mesh=mesh,
        compiler_params=pltpu.CompilerParams(needs_layout_passes=False,
                                             use_tc_tiling_on_sc=False),
        out_type=[jax.ShapeDtypeStruct((_B_ROWS, _E_DIM), jnp.float32),
                  jax.ShapeDtypeStruct((_N_E,), jnp.float32),
                  jax.ShapeDtypeStruct((_NW, 16), jnp.float32),
                  jax.ShapeDtypeStruct((16,), jnp.float32)],
        scratch_types=[pltpu.VMEM((_BPW,), jnp.int32),
                       pltpu.VMEM((_BPW, _E_DIM), jnp.float32),
                       pltpu.VMEM((_BPW, _E_DIM), jnp.float32),
                       pltpu.VMEM((16,), jnp.float32),
                       pltpu.VMEM((_B_ROWS,), jnp.int32),
                       pltpu.VMEM((_N_E,), jnp.float32),
                       pltpu.VMEM((16,), jnp.float32),
                       pltpu.SemaphoreType.DMA],
    )
    def sc_kernel(idx_hbm, emb_hbm, zp_hbm,
                  zq_hbm, usage_hbm, lp_hbm, up_hbm,
                  idx_v, rows_v, zp_v, acc_v, idxall_v, usage_v, uacc_v, sem):
        wid = lax.axis_index("s") * 2 + lax.axis_index("c")
        base = wid * _BPW
        pltpu.sync_copy(idx_hbm.at[pl.ds(base, _BPW)], idx_v)
        for g in range(_BPW // _GCH):
            pltpu.async_copy(
                emb_hbm.at[idx_v.at[pl.ds(g * _GCH, _GCH)]],
                rows_v.at[pl.ds(g * _GCH, _GCH)], sem)
        pltpu.sync_copy(zp_hbm.at[pl.ds(base, _BPW)], zp_v)
        for g in range(_BPW // _GCH):
            pltpu.make_async_copy(
                emb_hbm.at[idx_v.at[pl.ds(g * _GCH, _GCH)]],
                rows_v.at[pl.ds(g * _GCH, _GCH)], sem).wait()

        def row_step(i, acc):
            for h in range(_E_DIM // 16):
                g = rows_v[i, pl.ds(h * 16, 16)]
                p = zp_v[i, pl.ds(h * 16, 16)]
                df = g - p
                acc = acc + df * df
                rows_v[i, pl.ds(h * 16, 16)] = p + df
            return acc

        acc = lax.fori_loop(0, _BPW, row_step, jnp.zeros((16,), jnp.float32))
        acc_v[...] = acc
        pltpu.sync_copy(rows_v, zq_hbm.at[pl.ds(base, _BPW)])
        pltpu.sync_copy(acc_v, lp_hbm.at[wid])

        @pl.when(wid == 0)
        def _():
            pltpu.sync_copy(idx_hbm, idxall_v)

            def z_step(i, _):
                usage_v[pl.ds(i * 16, 16)] = jnp.zeros((16,), jnp.float32)
                return 0
            lax.fori_loop(0, _N_E // 16, z_step, 0)

            ones = jnp.ones((16,), jnp.float32)

            def s_step(j, _):
                iv = idxall_v[pl.ds(j * 16, 16)]
                plsc.store_scatter(usage_v, [iv], ones)
                return 0
            lax.fori_loop(0, _B_ROWS // 16, s_step, 0)

            def u_step(i, uacc):
                return uacc + usage_v[pl.ds(i * 16, 16)]
            uacc = lax.fori_loop(0, _N_E // 16, u_step,
                                 jnp.zeros((16,), jnp.float32))
            uacc_v[...] = uacc
            pltpu.sync_copy(usage_v, usage_hbm)
            pltpu.sync_copy(uacc_v, up_hbm)

    return sc_kernel


_sc_kernel = _make_sc_kernel()


def kernel(z, emb):
    zf = z.transpose(0, 2, 3, 1).reshape(_B_ROWS, _E_DIM)

    idx2d = _argmin_indices(zf, emb)
    idx = idx2d.reshape(_B_ROWS)

    zq_flat, usage, lp, up = _sc_kernel(idx, emb, zf)

    z_q_out = zq_flat.reshape(16, 32, 32, _E_DIM).transpose(0, 3, 1, 2)
    m = jnp.sum(lp) / jnp.float32(_B_ROWS * _E_DIM)
    loss = m + jnp.float32(_BETA) * m
    uni = jnp.sum(up)
    num_unique = uni.astype(jnp.int32)
    total_usage = uni / jnp.float32(_N_E)
    return (z_q_out, loss, idx, num_unique, usage, total_usage)
